# Initial kernel scaffold; baseline (speedup 1.0000x reference)
import jax
import jax.numpy as jnp
from jax.experimental import pallas as pl


def kernel(proposals, labels):
    raise NotImplementedError("write your pallas kernel here")


# speedup vs baseline: 28.9709x; 28.9709x over previous
import functools
import jax
import jax.numpy as jnp
from jax import lax
from jax.experimental import pallas as pl
from jax.experimental.pallas import tpu as pltpu

IOU_THRS = (0.5, 0.75)
BIG = 1e9
TP_ROWS = 16
TP_LANES = 128
TP_CAP = TP_ROWS * TP_LANES
CH = 256


def _ap_reduce(tc_ref, ti_ref, r_ref, p_ref, conff_ref, *, n_lab, n_flat):
    neg1 = jnp.float32(-1.0)
    tc3 = tc_ref[...].reshape(TP_ROWS, TP_LANES, 1)
    ti3 = ti_ref[...].reshape(TP_ROWS, TP_LANES, 1)

    def rbody(c, acc):
        cf = conff_ref[:, pl.ds(c * CH, CH)].reshape(1, 1, CH)
        jf = (lax.broadcasted_iota(jnp.int32, (1, 1, CH), 2)
              + c * CH).astype(jnp.float32)
        gt = cf > tc3
        tie = (cf == tc3) & (jf < ti3)
        return acc + jnp.sum((gt | tie).astype(jnp.float32), axis=2)

    r = lax.fori_loop(0, n_flat // CH, rbody,
                      jnp.ones((TP_ROWS, TP_LANES), dtype=jnp.float32))
    r_ref[...] = r
    r3 = r.reshape(TP_ROWS, TP_LANES, 1)

    def cbody(q, acc):
        rq = r_ref[pl.ds(q, 1), :].reshape(1, 1, TP_LANES)
        vq = (tc_ref[pl.ds(q, 1), :] > jnp.float32(-0.5)).reshape(1, 1, TP_LANES)
        return acc + jnp.sum((vq & (rq < r3)).astype(jnp.float32), axis=2)

    c = lax.fori_loop(0, TP_ROWS, cbody,
                      jnp.ones((TP_ROWS, TP_LANES), dtype=jnp.float32))
    p_ref[...] = c / r

    def mbody(q, acc):
        rq = r_ref[pl.ds(q, 1), :].reshape(1, 1, TP_LANES)
        vq = (tc_ref[pl.ds(q, 1), :] > jnp.float32(-0.5)).reshape(1, 1, TP_LANES)
        pq = p_ref[pl.ds(q, 1), :].reshape(1, 1, TP_LANES)
        return jnp.maximum(acc, jnp.max(jnp.where(vq & (rq >= r3), pq, neg1),
                                        axis=2))

    M = lax.fori_loop(0, TP_ROWS, mbody,
                      jnp.full((TP_ROWS, TP_LANES), neg1, dtype=jnp.float32))
    valid = tc_ref[...] > jnp.float32(-0.5)
    take = valid & (r >= jnp.float32(2.0))
    return jnp.sum(jnp.where(take, M, jnp.float32(0.0))) / jnp.float32(n_lab)


def _ap_body(pb_ref, pe_ref, conf_ref, conff_ref, labels_ref, out_ref,
             used5_ref, used7_ref, tc5_ref, ti5_ref, tc7_ref, ti7_ref,
             r5_ref, p5_ref, r7_ref, p7_ref, *,
             n_cols_real, n_lab, n_flat):
    pb = pb_ref[...]
    pe = pe_ref[...]
    conf2d = conf_ref[...]
    rows, n_cols = pb.shape
    row_i = lax.broadcasted_iota(jnp.int32, (rows, n_cols), 0)
    col_i = lax.broadcasted_iota(jnp.int32, (rows, n_cols), 1)
    gidx = (row_i * n_cols_real + col_i).astype(jnp.float32)

    sr = lax.broadcasted_iota(jnp.int32, (TP_ROWS, TP_LANES), 0)
    sc = lax.broadcasted_iota(jnp.int32, (TP_ROWS, TP_LANES), 1)
    slot = sr * TP_LANES + sc
    neg1 = jnp.float32(-1.0)

    used5_ref[...] = jnp.zeros(pb.shape, dtype=jnp.float32)
    used7_ref[...] = jnp.zeros(pb.shape, dtype=jnp.float32)
    tc5_ref[...] = jnp.full((TP_ROWS, TP_LANES), neg1, dtype=jnp.float32)
    ti5_ref[...] = jnp.full((TP_ROWS, TP_LANES), neg1, dtype=jnp.float32)
    tc7_ref[...] = jnp.full((TP_ROWS, TP_LANES), neg1, dtype=jnp.float32)
    ti7_ref[...] = jnp.full((TP_ROWS, TP_LANES), neg1, dtype=jnp.float32)

    def body(i, carry):
        cnt5, cnt7 = carry
        tb = labels_ref[i, 0]
        te = labels_ref[i, 1]
        inner = jnp.maximum(jnp.minimum(pe, te) - jnp.maximum(pb, tb), 0.0)
        outer = jnp.maximum(pe, te) - jnp.minimum(pb, tb)

        def step(thr, used_ref, cnt, tc_ref, ti_ref):
            used = used_ref[...]
            cand = (inner > thr * outer) & (used == 0.0)
            m = jnp.min(jnp.where(cand, gidx, BIG))
            hit = gidx == m
            used_ref[...] = jnp.where(hit, 1.0, used)
            has = m < BIG
            cm = jnp.max(jnp.where(hit, conf2d, neg1))
            sel = has & (slot == cnt)
            tc_ref[...] = jnp.where(sel, cm, tc_ref[...])
            ti_ref[...] = jnp.where(sel, m, ti_ref[...])
            return cnt + has.astype(jnp.int32)

        cnt5 = step(jnp.float32(IOU_THRS[0]), used5_ref, cnt5, tc5_ref, ti5_ref)
        cnt7 = step(jnp.float32(IOU_THRS[1]), used7_ref, cnt7, tc7_ref, ti7_ref)
        return cnt5, cnt7

    zero = jnp.int32(0)
    lax.fori_loop(0, n_lab, body, (zero, zero))

    ap5 = _ap_reduce(tc5_ref, ti5_ref, r5_ref, p5_ref, conff_ref,
                     n_lab=n_lab, n_flat=n_flat)
    ap7 = _ap_reduce(tc7_ref, ti7_ref, r7_ref, p7_ref, conff_ref,
                     n_lab=n_lab, n_flat=n_flat)

    lane = lax.broadcasted_iota(jnp.int32, (8, 128), 1)
    sub = lax.broadcasted_iota(jnp.int32, (8, 128), 0)
    out = jnp.where((sub == 0) & (lane == 0), ap5,
                    jnp.where((sub == 0) & (lane == 1), ap7, jnp.float32(0.0)))
    out_ref[...] = out


def kernel(proposals, labels):
    n_prop = proposals.shape[0]
    n_lab = labels.shape[0]
    rows = 8
    n_cols_real = -(-n_prop // rows)
    n_cols = -(-n_cols_real // 128) * 128
    n_flat = -(-(rows * n_cols_real) // CH) * CH

    conf = proposals[:, 0]
    pb = proposals[:, 1]
    pe = proposals[:, 2]
    pad2d = rows * n_cols_real - n_prop

    def to2d(x, fill):
        x = jnp.concatenate([x, jnp.full((pad2d,), fill, x.dtype)]) if pad2d else x
        x = x.reshape(rows, n_cols_real)
        if n_cols != n_cols_real:
            x = jnp.pad(x, ((0, 0), (0, n_cols - n_cols_real)),
                        constant_values=fill)
        return x

    conf2d = to2d(conf, -1.0)
    pb2d = to2d(pb, -1.0)
    pe2d = to2d(pe, -1.0)
    conff = jnp.pad(conf, (0, n_flat - n_prop), constant_values=-1.0)[None, :]

    out2d = pl.pallas_call(
        functools.partial(_ap_body, n_cols_real=n_cols_real, n_lab=n_lab,
                          n_flat=n_flat),
        in_specs=[
            pl.BlockSpec(memory_space=pltpu.VMEM),
            pl.BlockSpec(memory_space=pltpu.VMEM),
            pl.BlockSpec(memory_space=pltpu.VMEM),
            pl.BlockSpec(memory_space=pltpu.VMEM),
            pl.BlockSpec(memory_space=pltpu.SMEM),
        ],
        out_specs=pl.BlockSpec(memory_space=pltpu.VMEM),
        out_shape=jax.ShapeDtypeStruct((8, 128), jnp.float32),
        scratch_shapes=[
            pltpu.VMEM((rows, n_cols), jnp.float32),
            pltpu.VMEM((rows, n_cols), jnp.float32),
            pltpu.VMEM((TP_ROWS, TP_LANES), jnp.float32),
            pltpu.VMEM((TP_ROWS, TP_LANES), jnp.float32),
            pltpu.VMEM((TP_ROWS, TP_LANES), jnp.float32),
            pltpu.VMEM((TP_ROWS, TP_LANES), jnp.float32),
            pltpu.VMEM((TP_ROWS, TP_LANES), jnp.float32),
            pltpu.VMEM((TP_ROWS, TP_LANES), jnp.float32),
            pltpu.VMEM((TP_ROWS, TP_LANES), jnp.float32),
            pltpu.VMEM((TP_ROWS, TP_LANES), jnp.float32),
        ],
    )(pb2d, pe2d, conf2d, conff, labels)
    return out2d[0, :2]
